# baseline (speedup 1.0000x reference)
import jax
import jax.numpy as jnp
from jax.experimental import pallas as pl
from jax.experimental.pallas import tpu as pltpu

_N = 20000
_K = 8
_M = 1000
_MP = 1024
_NC = 125
_L = 1280
_TOPK = 100
_PP = 128
_IMG_W = 512.0
_IMG_H = 512.0
_SCORE_THRESH = 0.05
_NMS_THRESH = 0.5


def _mega_kernel(fg_ref, tab_ref, out_ref, a_ref, cb_ref, vf_ref):
    f32 = jnp.float32
    i32 = jnp.int32

    fg = fg_ref[...]
    thr = fg > _SCORE_THRESH
    vb = jax.lax.bitcast_convert_type(jnp.where(thr, fg, 0.0), i32)

    def bis_body(_, lohi):
        lo, hi = lohi
        mid = lo + (hi - lo) // 2
        cnt = jnp.sum(jnp.where(vb >= mid, 1.0, 0.0))
        ge = cnt >= float(_M)
        return jnp.where(ge, mid, lo), jnp.where(ge, hi, mid)

    lo, _ = jax.lax.fori_loop(0, 31, bis_body,
                              (jnp.int32(1), jnp.int32(0x7F800000)))
    mask = jnp.where(vb >= lo, 1.0, 0.0)
    nsurv = jnp.sum(mask)

    li_t = jax.lax.broadcasted_iota(i32, (_L, _L), 1)
    si_t = jax.lax.broadcasted_iota(i32, (_L, _L), 0)
    tri_excl = jnp.where(si_t < li_t, 1.0, 0.0).astype(jnp.bfloat16)
    r_in = jnp.dot(mask.astype(jnp.bfloat16), tri_excl,
                   preferred_element_type=f32)
    cnt_col = jnp.sum(mask, axis=1, keepdims=True)
    li_c = jax.lax.broadcasted_iota(i32, (_NC, _NC), 1)
    si_c = jax.lax.broadcasted_iota(i32, (_NC, _NC), 0)
    base_row = jnp.sum(jnp.where(si_c < li_c, 1.0, 0.0) * cnt_col,
                       axis=0, keepdims=True)
    base_col = jnp.sum(jnp.where(li_c == si_c, 1.0, 0.0) * base_row,
                       axis=1, keepdims=True)
    slot = base_col + r_in
    li_f = jax.lax.broadcasted_iota(i32, (_NC, _L), 1)
    si_f = jax.lax.broadcasted_iota(i32, (_NC, _L), 0)
    pos = si_f * _L + li_f
    rem = jnp.where(mask > 0.0, pos - slot.astype(i32), 0)

    fidx = pos

    def flat_roll(x, d):
        dl = d % _L
        dr = d // _L
        a = jnp.concatenate([x[:, dl:], x[:, :dl]], axis=1)
        b = a if dr == 0 else jnp.concatenate([a[dr:], a[:dr]], axis=0)
        c = jnp.concatenate([a[dr + 1:], a[:dr + 1]], axis=0)
        return jnp.where(li_f < _L - dl, b, c)

    for d in [1, 2, 4, 8, 16, 32, 64, 128, 256, 512, 1024, 2048, 4096,
              8192, 16384, 32768, 65536, 131072]:
        rr = flat_roll(rem, d)
        rf = flat_roll(fidx, d)
        sel = (rr & d) != 0
        fidx = jnp.where(sel, rf, fidx)
        rem = jnp.where(sel, rr - d, rem)

    fidx_row = fidx[0:1, 0:_MP].astype(f32)

    lim = jax.lax.broadcasted_iota(i32, (_MP, _MP), 1)
    sim = jax.lax.broadcasted_iota(i32, (_MP, _MP), 0)
    eye = jnp.where(lim == sim, 1.0, 0.0)

    def to_col(row):
        return jnp.sum(eye * row, axis=1, keepdims=True)

    def to_row(col):
        return jnp.sum(eye * col, axis=0, keepdims=True)

    fidx_col = to_col(fidx_row)
    vf_ref[:, 1:2] = fidx_col
    li8 = jax.lax.broadcasted_iota(i32, (1, 8), 1).astype(f32)
    li32 = jax.lax.broadcasted_iota(i32, (1, 32), 1).astype(f32)

    def gather_body(s, carry):
        fx = jnp.sum(vf_ref[pl.ds(s, 1), 1:2])
        bi = jnp.floor(fx * 0.125)
        cls = fx - 8.0 * bi
        trow = tab_ref[pl.ds(bi.astype(i32), 1), :]
        sval = jnp.sum(trow[:, 0:8] * jnp.where(li8 == cls, 1.0, 0.0))
        rel = li32 - 4.0 * cls
        box = jnp.concatenate(
            [jnp.sum(trow[:, 8:40] * jnp.where(rel == float(j), 1.0, 0.0),
                     keepdims=True).reshape(1, 1) for j in range(4)],
            axis=1)
        cb_ref[pl.ds(s, 1), :] = jnp.clip(box, 0.0, _IMG_W)
        vf_ref[pl.ds(s, 1), 0:1] = jnp.zeros((1, 1), f32) + sval
        return carry

    jax.lax.fori_loop(0, _MP, gather_body, 0)

    val_col = vf_ref[:, 0:1]
    val_row = to_row(val_col)
    x1c = cb_ref[:, 0:1]
    y1c = cb_ref[:, 1:2]
    x2c = cb_ref[:, 2:3]
    y2c = cb_ref[:, 3:4]
    x1r, y1r, x2r, y2r = map(to_row, (x1c, y1c, x2c, y2c))

    bi_col = jnp.floor(fidx_col * 0.125)
    cls_col = fidx_col - 8.0 * bi_col
    cls_row = fidx_row - 8.0 * jnp.floor(fidx_row * 0.125)
    off = max(_IMG_W, _IMG_H) + 1.0
    ox1c, oy1c, ox2c, oy2c = (x1c + cls_col * off, y1c + cls_col * off,
                              x2c + cls_col * off, y2c + cls_col * off)
    ox1r, oy1r, ox2r, oy2r = (x1r + cls_row * off, y1r + cls_row * off,
                              x2r + cls_row * off, y2r + cls_row * off)

    area_c = jnp.maximum(ox2c - ox1c, 0.0) * jnp.maximum(oy2c - oy1c, 0.0)
    area_r = jnp.maximum(ox2r - ox1r, 0.0) * jnp.maximum(oy2r - oy1r, 0.0)
    iw = jnp.maximum(jnp.minimum(ox2c, ox2r) - jnp.maximum(ox1c, ox1r), 0.0)
    ih = jnp.maximum(jnp.minimum(oy2c, oy2r) - jnp.maximum(oy1c, oy1r), 0.0)
    inter = iw * ih
    iou = inter / jnp.maximum(area_c + area_r - inter, 1e-9)
    pgt = jnp.where((val_col > val_row)
                    | ((val_col == val_row) & (fidx_col < fidx_row)),
                    1.0, 0.0)
    a_ref[...] = jnp.where(iou > _NMS_THRESH, 1.0, 0.0) * pgt

    li1 = jax.lax.broadcasted_iota(i32, (1, _MP), 1).astype(f32)
    valid0 = jnp.where(li1 < nsurv, 1.0, 0.0)
    prank = jnp.sum(pgt * to_col(valid0), axis=0, keepdims=True)
    valid = valid0 * jnp.where(prank < float(_M), 1.0, 0.0)

    def nms_round(state):
        kept, und = state
        threat = jnp.sum(a_ref[...] * to_col(kept + und),
                         axis=0, keepdims=True)
        kthreat = jnp.sum(a_ref[...] * to_col(kept),
                          axis=0, keepdims=True)
        new_kept = und * jnp.where(threat == 0.0, 1.0, 0.0)
        new_dead = und * jnp.where(kthreat > 0.0, 1.0, 0.0)
        return kept + new_kept, und * (1.0 - new_kept) * (1.0 - new_dead)

    keep, _ = jax.lax.while_loop(lambda st: jnp.sum(st[1]) > 0.0, nms_round,
                                 (jnp.zeros_like(valid), valid))

    r_row = jnp.sum(pgt * to_col(keep), axis=0, keepdims=True)
    n_row = jnp.sum(pgt * to_col(valid - keep), axis=0, keepdims=True)
    tkept = jnp.sum(keep)
    osel = jnp.where(keep > 0.0, r_row,
                     jnp.where(valid > 0.0, tkept + n_row, 1e9))
    p_sub = jax.lax.broadcasted_iota(i32, (_PP, _MP), 0).astype(f32)
    p_lan = jax.lax.broadcasted_iota(i32, (_PP, _MP), 1).astype(f32)
    idx_sel = jnp.sum(jnp.where(osel == p_sub, 1.0, 0.0) * p_lan,
                      axis=1, keepdims=True)
    sub_pp = jax.lax.broadcasted_iota(i32, (_PP, 1), 0)

    def out_body(p, carry):
        ip = jnp.sum(idx_sel * jnp.where(sub_pp == p, 1.0, 0.0)
                     ).astype(i32)
        cbrow = cb_ref[pl.ds(ip, 1), :]
        vfrow = vf_ref[pl.ds(ip, 1), :]
        scrow = vfrow[:, 0:1]
        bi = jnp.floor(jnp.sum(vfrow[:, 1:2]) * 0.125).astype(i32)
        mrow = tab_ref[pl.ds(bi, 1), 40:61]
        out_ref[pl.ds(p, 1), :] = jnp.concatenate([cbrow, scrow, mrow],
                                                  axis=1)
        return carry

    jax.lax.fori_loop(0, _TOPK, out_body, 0)


def kernel(boxes, scores, mtype, morigin, maxis, mextrinsic):
    f32 = jnp.float32
    fgm = scores[:, :-1].reshape(_NC, _L).astype(f32)
    table = jnp.concatenate([scores[:, :-1], boxes, mtype, morigin,
                             maxis, mextrinsic], axis=1)
    return pl.pallas_call(
        _mega_kernel,
        out_shape=jax.ShapeDtypeStruct((_TOPK, 26), f32),
        scratch_shapes=[pltpu.VMEM((_MP, _MP), f32),
                        pltpu.VMEM((_MP, 4), f32),
                        pltpu.VMEM((_MP, 2), f32)],
    )(fgm, table)


# speedup vs baseline: 3.1022x; 3.1022x over previous
import jax
import jax.numpy as jnp
from jax.experimental import pallas as pl
from jax.experimental.pallas import tpu as pltpu

_N = 20000
_K = 8
_M = 1000
_MP = 1024
_NC = 125
_L = 1280
_TOPK = 100
_PP = 128
_IMG_W = 512.0
_IMG_H = 512.0
_SCORE_THRESH = 0.05
_NMS_THRESH = 0.5


def _mega_kernel(fg_ref, x1_ref, y1_ref, x2_ref, y2_ref, motion_ref,
                 out_ref, a_ref, cb_ref, vf_ref):
    f32 = jnp.float32
    i32 = jnp.int32

    fg = fg_ref[...]
    thr = fg > _SCORE_THRESH
    vb = jax.lax.bitcast_convert_type(jnp.where(thr, fg, 0.0), i32)

    def bis_body(_, lohi):
        lo, hi = lohi
        mid = lo + (hi - lo) // 2
        cnt = jnp.sum(jnp.where(vb >= mid, 1.0, 0.0))
        ge = cnt >= float(_M)
        return jnp.where(ge, mid, lo), jnp.where(ge, hi, mid)

    lo, _ = jax.lax.fori_loop(0, 31, bis_body,
                              (jnp.int32(1), jnp.int32(0x7F800000)))
    mask = jnp.where(vb >= lo, 1.0, 0.0)
    nsurv = jnp.sum(mask)

    li_t = jax.lax.broadcasted_iota(i32, (_L, _L), 1)
    si_t = jax.lax.broadcasted_iota(i32, (_L, _L), 0)
    tri_excl = jnp.where(si_t < li_t, 1.0, 0.0).astype(jnp.bfloat16)
    r_in = jnp.dot(mask.astype(jnp.bfloat16), tri_excl,
                   preferred_element_type=f32)
    cnt_col = jnp.sum(mask, axis=1, keepdims=True)
    li_c = jax.lax.broadcasted_iota(i32, (_NC, _NC), 1)
    si_c = jax.lax.broadcasted_iota(i32, (_NC, _NC), 0)
    base_row = jnp.sum(jnp.where(si_c < li_c, 1.0, 0.0) * cnt_col,
                       axis=0, keepdims=True)
    base_col = jnp.sum(jnp.where(li_c == si_c, 1.0, 0.0) * base_row,
                       axis=1, keepdims=True)
    li_f = jax.lax.broadcasted_iota(i32, (_NC, _L), 1)
    si_f = jax.lax.broadcasted_iota(i32, (_NC, _L), 0)
    pos = si_f * _L + li_f
    rem = jnp.where(mask > 0.0, li_f - r_in.astype(i32), 0)

    arrs = [jnp.where(thr, fg, 0.0), pos,
            x1_ref[...], y1_ref[...], x2_ref[...], y2_ref[...]]

    def lshift(x, d):
        return jnp.concatenate([x[:, d:], x[:, :d]], axis=1)

    for d in [1, 2, 4, 8, 16, 32, 64, 128, 256, 512, 1024]:
        rr = lshift(rem, d)
        sel = (rr & d) != 0
        arrs = [jnp.where(sel, lshift(x, d), x) for x in arrs]
        rem = jnp.where(sel, rr - d, rem)

    base_i = base_col.astype(i32)
    cnt_i = cnt_col.astype(i32)

    def rshift(x, d):
        return jnp.concatenate([x[:, _L - d:], x[:, :_L - d]], axis=1)

    for d in [1, 2, 4, 8, 16, 32, 64, 128, 256, 512, 1024]:
        selr = (base_i & d) != 0
        arrs = [jnp.where(selr, rshift(x, d), x) for x in arrs]

    seg = (li_f >= base_i) & (li_f < base_i + cnt_i)
    merged = [jnp.sum(jnp.where(seg, x, x - x), axis=0, keepdims=True)
              for x in arrs]
    val_row = merged[0][0:1, 0:_MP]
    fidx_row = merged[1][0:1, 0:_MP].astype(f32)
    x1r = jnp.clip(merged[2][0:1, 0:_MP], 0.0, _IMG_W)
    y1r = jnp.clip(merged[3][0:1, 0:_MP], 0.0, _IMG_H)
    x2r = jnp.clip(merged[4][0:1, 0:_MP], 0.0, _IMG_W)
    y2r = jnp.clip(merged[5][0:1, 0:_MP], 0.0, _IMG_H)

    lim = jax.lax.broadcasted_iota(i32, (_MP, _MP), 1)
    sim = jax.lax.broadcasted_iota(i32, (_MP, _MP), 0)
    eye = jnp.where(lim == sim, 1.0, 0.0)

    def to_col(row):
        return jnp.sum(eye * row, axis=1, keepdims=True)

    val_col = to_col(val_row)
    fidx_col = to_col(fidx_row)
    x1c, y1c, x2c, y2c = map(to_col, (x1r, y1r, x2r, y2r))
    vf_ref[:, 0:1] = val_col
    vf_ref[:, 1:2] = fidx_col
    cb_ref[:, 0:1] = x1c
    cb_ref[:, 1:2] = y1c
    cb_ref[:, 2:3] = x2c
    cb_ref[:, 3:4] = y2c

    bi_col = jnp.floor(fidx_col * 0.125)
    cls_col = fidx_col - 8.0 * bi_col
    cls_row = fidx_row - 8.0 * jnp.floor(fidx_row * 0.125)
    off = max(_IMG_W, _IMG_H) + 1.0
    ox1c, oy1c, ox2c, oy2c = (x1c + cls_col * off, y1c + cls_col * off,
                              x2c + cls_col * off, y2c + cls_col * off)
    ox1r, oy1r, ox2r, oy2r = (x1r + cls_row * off, y1r + cls_row * off,
                              x2r + cls_row * off, y2r + cls_row * off)

    area_c = jnp.maximum(ox2c - ox1c, 0.0) * jnp.maximum(oy2c - oy1c, 0.0)
    area_r = jnp.maximum(ox2r - ox1r, 0.0) * jnp.maximum(oy2r - oy1r, 0.0)
    iw = jnp.maximum(jnp.minimum(ox2c, ox2r) - jnp.maximum(ox1c, ox1r), 0.0)
    ih = jnp.maximum(jnp.minimum(oy2c, oy2r) - jnp.maximum(oy1c, oy1r), 0.0)
    inter = iw * ih
    iou = inter / jnp.maximum(area_c + area_r - inter, 1e-9)
    pgt = jnp.where((val_col > val_row)
                    | ((val_col == val_row) & (fidx_col < fidx_row)),
                    1.0, 0.0)
    a_ref[...] = jnp.where(iou > _NMS_THRESH, 1.0, 0.0) * pgt

    li1 = jax.lax.broadcasted_iota(i32, (1, _MP), 1).astype(f32)
    valid0 = jnp.where(li1 < nsurv, 1.0, 0.0)
    prank = jnp.sum(pgt * to_col(valid0), axis=0, keepdims=True)
    valid = valid0 * jnp.where(prank < float(_M), 1.0, 0.0)

    def nms_round(state):
        kept, und = state
        threat = jnp.sum(a_ref[...] * to_col(kept + und),
                         axis=0, keepdims=True)
        kthreat = jnp.sum(a_ref[...] * to_col(kept),
                          axis=0, keepdims=True)
        new_kept = und * jnp.where(threat == 0.0, 1.0, 0.0)
        new_dead = und * jnp.where(kthreat > 0.0, 1.0, 0.0)
        return kept + new_kept, und * (1.0 - new_kept) * (1.0 - new_dead)

    keep, _ = jax.lax.while_loop(lambda st: jnp.sum(st[1]) > 0.0, nms_round,
                                 (jnp.zeros_like(valid), valid))

    r_row = jnp.sum(pgt * to_col(keep), axis=0, keepdims=True)
    n_row = jnp.sum(pgt * to_col(valid - keep), axis=0, keepdims=True)
    tkept = jnp.sum(keep)
    osel = jnp.where(keep > 0.0, r_row,
                     jnp.where(valid > 0.0, tkept + n_row, 1e9))
    p_sub = jax.lax.broadcasted_iota(i32, (_PP, _MP), 0).astype(f32)
    p_lan = jax.lax.broadcasted_iota(i32, (_PP, _MP), 1).astype(f32)
    idx_sel = jnp.sum(jnp.where(osel == p_sub, 1.0, 0.0) * p_lan,
                      axis=1, keepdims=True)
    sub_pp = jax.lax.broadcasted_iota(i32, (_PP, 1), 0)

    def out_body(p, carry):
        ip = jnp.sum(idx_sel * jnp.where(sub_pp == p, 1.0, 0.0)
                     ).astype(i32)
        cbrow = cb_ref[pl.ds(ip, 1), :]
        vfrow = vf_ref[pl.ds(ip, 1), :]
        scrow = vfrow[:, 0:1]
        bi = jnp.floor(jnp.sum(vfrow[:, 1:2]) * 0.125).astype(i32)
        mrow = motion_ref[pl.ds(bi, 1), :]
        out_ref[pl.ds(p, 1), :] = jnp.concatenate([cbrow, scrow, mrow],
                                                  axis=1)
        return carry

    jax.lax.fori_loop(0, _TOPK, out_body, 0)


def kernel(boxes, scores, mtype, morigin, maxis, mextrinsic):
    f32 = jnp.float32
    fgm = scores[:, :-1].reshape(_NC, _L).astype(f32)
    bc = boxes.reshape(_N, _K, 4)
    planes = [bc[:, :, c].reshape(_NC, _L).astype(f32) for c in range(4)]
    motion = jnp.concatenate([mtype, morigin, maxis, mextrinsic], axis=1)
    return pl.pallas_call(
        _mega_kernel,
        out_shape=jax.ShapeDtypeStruct((_TOPK, 26), f32),
        scratch_shapes=[pltpu.VMEM((_MP, _MP), f32),
                        pltpu.VMEM((_MP, 4), f32),
                        pltpu.VMEM((_MP, 2), f32)],
    )(fgm, *planes, motion)


# speedup vs baseline: 3.1498x; 1.0154x over previous
import jax
import jax.numpy as jnp
from jax.experimental import pallas as pl
from jax.experimental.pallas import tpu as pltpu

_N = 20000
_K = 8
_M = 1000
_MP = 1024
_NC = 125
_L = 1280
_TOPK = 100
_PP = 128
_IMG_W = 512.0
_IMG_H = 512.0
_SCORE_THRESH = 0.05
_NMS_THRESH = 0.5


def _mega_kernel(fg_ref, x1_ref, y1_ref, x2_ref, y2_ref, motion_ref,
                 out_ref, a_ref, cb_ref, vf_ref):
    f32 = jnp.float32
    i32 = jnp.int32

    fg = fg_ref[...]
    thr = fg > _SCORE_THRESH
    vb = jax.lax.bitcast_convert_type(jnp.where(thr, fg, 0.0), i32)

    def bis_body(_, lohi):
        lo, hi = lohi
        step = jnp.maximum((hi - lo) // 8, 1)
        new_lo, new_hi = lo, hi
        for k in range(1, 8):
            mk = lo + step * k
            ge = jnp.sum(jnp.where(vb >= mk, 1.0, 0.0)) >= float(_M)
            new_lo = jnp.where(ge, jnp.maximum(new_lo, mk), new_lo)
            new_hi = jnp.where(ge, new_hi, jnp.minimum(new_hi, mk))
        return new_lo, new_hi

    lo, _ = jax.lax.fori_loop(0, 11, bis_body,
                              (jnp.int32(1), jnp.int32(0x7F800000)))
    mask = jnp.where(vb >= lo, 1.0, 0.0)
    nsurv = jnp.sum(mask)

    li_t = jax.lax.broadcasted_iota(i32, (_L, _L), 1)
    si_t = jax.lax.broadcasted_iota(i32, (_L, _L), 0)
    tri_excl = jnp.where(si_t < li_t, 1.0, 0.0).astype(jnp.bfloat16)
    r_in = jnp.dot(mask.astype(jnp.bfloat16), tri_excl,
                   preferred_element_type=f32)
    cnt_col = jnp.sum(mask, axis=1, keepdims=True)
    li_c = jax.lax.broadcasted_iota(i32, (_NC, _NC), 1)
    si_c = jax.lax.broadcasted_iota(i32, (_NC, _NC), 0)
    base_row = jnp.sum(jnp.where(si_c < li_c, 1.0, 0.0) * cnt_col,
                       axis=0, keepdims=True)
    base_col = jnp.sum(jnp.where(li_c == si_c, 1.0, 0.0) * base_row,
                       axis=1, keepdims=True)
    li_f = jax.lax.broadcasted_iota(i32, (_NC, _L), 1)
    si_f = jax.lax.broadcasted_iota(i32, (_NC, _L), 0)
    pos = si_f * _L + li_f
    rem = jnp.where(mask > 0.0, li_f - r_in.astype(i32), 0)

    arrs = [jnp.where(thr, fg, 0.0), pos,
            x1_ref[...], y1_ref[...], x2_ref[...], y2_ref[...]]

    def lshift(x, d):
        return jnp.concatenate([x[:, d:], x[:, :d]], axis=1)

    for d in [1, 2, 4, 8, 16, 32, 64, 128, 256, 512, 1024]:
        rr = lshift(rem, d)
        sel = (rr & d) != 0
        arrs = [jnp.where(sel, lshift(x, d), x) for x in arrs]
        rem = jnp.where(sel, rr - d, rem)

    base_i = base_col.astype(i32)
    cnt_i = cnt_col.astype(i32)

    def rshift(x, d):
        return jnp.concatenate([x[:, _L - d:], x[:, :_L - d]], axis=1)

    for d in [1, 2, 4, 8, 16, 32, 64, 128, 256, 512, 1024]:
        selr = (base_i & d) != 0
        arrs = [jnp.where(selr, rshift(x, d), x) for x in arrs]

    seg = (li_f >= base_i) & (li_f < base_i + cnt_i)
    merged = [jnp.sum(jnp.where(seg, x, x - x), axis=0, keepdims=True)
              for x in arrs]
    val_row = merged[0][0:1, 0:_MP]
    fidx_row = merged[1][0:1, 0:_MP].astype(f32)
    x1r = jnp.clip(merged[2][0:1, 0:_MP], 0.0, _IMG_W)
    y1r = jnp.clip(merged[3][0:1, 0:_MP], 0.0, _IMG_H)
    x2r = jnp.clip(merged[4][0:1, 0:_MP], 0.0, _IMG_W)
    y2r = jnp.clip(merged[5][0:1, 0:_MP], 0.0, _IMG_H)

    lim = jax.lax.broadcasted_iota(i32, (_MP, _MP), 1)
    sim = jax.lax.broadcasted_iota(i32, (_MP, _MP), 0)
    eye = jnp.where(lim == sim, 1.0, 0.0)

    def to_col(row):
        return jnp.sum(eye * row, axis=1, keepdims=True)

    val_col = to_col(val_row)
    fidx_col = to_col(fidx_row)
    x1c, y1c, x2c, y2c = map(to_col, (x1r, y1r, x2r, y2r))
    vf_ref[:, 0:1] = val_col
    vf_ref[:, 1:2] = fidx_col
    cb_ref[:, 0:1] = x1c
    cb_ref[:, 1:2] = y1c
    cb_ref[:, 2:3] = x2c
    cb_ref[:, 3:4] = y2c

    bi_col = jnp.floor(fidx_col * 0.125)
    cls_col = fidx_col - 8.0 * bi_col
    cls_row = fidx_row - 8.0 * jnp.floor(fidx_row * 0.125)
    off = max(_IMG_W, _IMG_H) + 1.0
    ox1c, oy1c, ox2c, oy2c = (x1c + cls_col * off, y1c + cls_col * off,
                              x2c + cls_col * off, y2c + cls_col * off)
    ox1r, oy1r, ox2r, oy2r = (x1r + cls_row * off, y1r + cls_row * off,
                              x2r + cls_row * off, y2r + cls_row * off)

    area_c = jnp.maximum(ox2c - ox1c, 0.0) * jnp.maximum(oy2c - oy1c, 0.0)
    area_r = jnp.maximum(ox2r - ox1r, 0.0) * jnp.maximum(oy2r - oy1r, 0.0)
    iw = jnp.maximum(jnp.minimum(ox2c, ox2r) - jnp.maximum(ox1c, ox1r), 0.0)
    ih = jnp.maximum(jnp.minimum(oy2c, oy2r) - jnp.maximum(oy1c, oy1r), 0.0)
    inter = iw * ih
    iou = inter / jnp.maximum(area_c + area_r - inter, 1e-9)
    pgt = jnp.where((val_col > val_row)
                    | ((val_col == val_row) & (fidx_col < fidx_row)),
                    1.0, 0.0)
    a_ref[...] = jnp.where(iou > _NMS_THRESH, 1.0, 0.0) * pgt

    li1 = jax.lax.broadcasted_iota(i32, (1, _MP), 1).astype(f32)
    valid0 = jnp.where(li1 < nsurv, 1.0, 0.0)
    prank = jnp.sum(pgt * to_col(valid0), axis=0, keepdims=True)
    valid = valid0 * jnp.where(prank < float(_M), 1.0, 0.0)

    def nms_round(state):
        kept, und = state
        threat = jnp.sum(a_ref[...] * to_col(kept + und),
                         axis=0, keepdims=True)
        kthreat = jnp.sum(a_ref[...] * to_col(kept),
                          axis=0, keepdims=True)
        new_kept = und * jnp.where(threat == 0.0, 1.0, 0.0)
        new_dead = und * jnp.where(kthreat > 0.0, 1.0, 0.0)
        return kept + new_kept, und * (1.0 - new_kept) * (1.0 - new_dead)

    keep, _ = jax.lax.while_loop(lambda st: jnp.sum(st[1]) > 0.0, nms_round,
                                 (jnp.zeros_like(valid), valid))

    r_row = jnp.sum(pgt * to_col(keep), axis=0, keepdims=True)
    n_row = jnp.sum(pgt * to_col(valid - keep), axis=0, keepdims=True)
    tkept = jnp.sum(keep)
    osel = jnp.where(keep > 0.0, r_row,
                     jnp.where(valid > 0.0, tkept + n_row, 1e9))
    p_sub = jax.lax.broadcasted_iota(i32, (_PP, _MP), 0).astype(f32)
    p_lan = jax.lax.broadcasted_iota(i32, (_PP, _MP), 1).astype(f32)
    idx_sel = jnp.sum(jnp.where(osel == p_sub, 1.0, 0.0) * p_lan,
                      axis=1, keepdims=True)
    sub_pp = jax.lax.broadcasted_iota(i32, (_PP, 1), 0)

    def out_body(p, carry):
        ip = jnp.sum(idx_sel * jnp.where(sub_pp == p, 1.0, 0.0)
                     ).astype(i32)
        cbrow = cb_ref[pl.ds(ip, 1), :]
        vfrow = vf_ref[pl.ds(ip, 1), :]
        scrow = vfrow[:, 0:1]
        bi = jnp.floor(jnp.sum(vfrow[:, 1:2]) * 0.125).astype(i32)
        mrow = motion_ref[pl.ds(bi, 1), :]
        out_ref[pl.ds(p, 1), :] = jnp.concatenate([cbrow, scrow, mrow],
                                                  axis=1)
        return carry

    jax.lax.fori_loop(0, _TOPK, out_body, 0)


def kernel(boxes, scores, mtype, morigin, maxis, mextrinsic):
    f32 = jnp.float32
    fgm = scores[:, :-1].reshape(_NC, _L).astype(f32)
    bc = boxes.reshape(_N, _K, 4)
    planes = [bc[:, :, c].reshape(_NC, _L).astype(f32) for c in range(4)]
    motion = jnp.concatenate([mtype, morigin, maxis, mextrinsic], axis=1)
    return pl.pallas_call(
        _mega_kernel,
        out_shape=jax.ShapeDtypeStruct((_TOPK, 26), f32),
        scratch_shapes=[pltpu.VMEM((_MP, _MP), f32),
                        pltpu.VMEM((_MP, 4), f32),
                        pltpu.VMEM((_MP, 2), f32)],
    )(fgm, *planes, motion)


# speedup vs baseline: 3.6059x; 1.1448x over previous
import jax
import jax.numpy as jnp
from jax.experimental import pallas as pl
from jax.experimental.pallas import tpu as pltpu

_N = 20000
_K = 8
_M = 1000
_MP = 1024
_NC = 125
_L = 1280
_TOPK = 100
_PP = 128
_IMG_W = 512.0
_IMG_H = 512.0
_SCORE_THRESH = 0.05
_NMS_THRESH = 0.5


def _mega_kernel(fg_ref, x1_ref, y1_ref, x2_ref, y2_ref, motion_ref,
                 out_ref, a_ref):
    f32 = jnp.float32
    i32 = jnp.int32

    fg = fg_ref[...]
    thr = fg > _SCORE_THRESH
    vb = jax.lax.bitcast_convert_type(jnp.where(thr, fg, 0.0), i32)

    def bis_body(_, lohi):
        lo, hi = lohi
        step = jnp.maximum((hi - lo) // 8, 1)
        new_lo, new_hi = lo, hi
        for k in range(1, 8):
            mk = lo + step * k
            ge = jnp.sum(jnp.where(vb >= mk, 1.0, 0.0)) >= float(_M)
            new_lo = jnp.where(ge, jnp.maximum(new_lo, mk), new_lo)
            new_hi = jnp.where(ge, new_hi, jnp.minimum(new_hi, mk))
        return new_lo, new_hi

    lo, _ = jax.lax.fori_loop(0, 11, bis_body,
                              (jnp.int32(1), jnp.int32(0x7F800000)))
    mask = jnp.where(vb >= lo, 1.0, 0.0)
    nsurv = jnp.sum(mask)

    li_t = jax.lax.broadcasted_iota(i32, (_L, _L), 1)
    si_t = jax.lax.broadcasted_iota(i32, (_L, _L), 0)
    tri_excl = jnp.where(si_t < li_t, 1.0, 0.0).astype(jnp.bfloat16)
    r_in = jnp.dot(mask.astype(jnp.bfloat16), tri_excl,
                   preferred_element_type=f32)
    cnt_col = jnp.sum(mask, axis=1, keepdims=True)
    li_c = jax.lax.broadcasted_iota(i32, (_NC, _NC), 1)
    si_c = jax.lax.broadcasted_iota(i32, (_NC, _NC), 0)
    base_row = jnp.sum(jnp.where(si_c < li_c, 1.0, 0.0) * cnt_col,
                       axis=0, keepdims=True)
    base_col = jnp.sum(jnp.where(li_c == si_c, 1.0, 0.0) * base_row,
                       axis=1, keepdims=True)
    li_f = jax.lax.broadcasted_iota(i32, (_NC, _L), 1)
    si_f = jax.lax.broadcasted_iota(i32, (_NC, _L), 0)
    pos = si_f * _L + li_f
    rem = jnp.where(mask > 0.0, li_f - r_in.astype(i32), 0)

    arrs = [jnp.where(thr, fg, 0.0), pos,
            x1_ref[...], y1_ref[...], x2_ref[...], y2_ref[...]]

    def lshift(x, d):
        return jnp.concatenate([x[:, d:], x[:, :d]], axis=1)

    for d in [1, 2, 4, 8, 16, 32, 64, 128, 256, 512, 1024]:
        rr = lshift(rem, d)
        sel = (rr & d) != 0
        arrs = [jnp.where(sel, lshift(x, d), x) for x in arrs]
        rem = jnp.where(sel, rr - d, rem)

    base_i = base_col.astype(i32)
    cnt_i = cnt_col.astype(i32)

    def rshift(x, d):
        return jnp.concatenate([x[:, _L - d:], x[:, :_L - d]], axis=1)

    for d in [1, 2, 4, 8, 16, 32, 64, 128, 256, 512, 1024]:
        selr = (base_i & d) != 0
        arrs = [jnp.where(selr, rshift(x, d), x) for x in arrs]

    seg = (li_f >= base_i) & (li_f < base_i + cnt_i)
    merged = [jnp.sum(jnp.where(seg, x, x - x), axis=0, keepdims=True)
              for x in arrs]
    val_row = merged[0][0:1, 0:_MP]
    fidx_row = merged[1][0:1, 0:_MP].astype(f32)
    x1r = jnp.clip(merged[2][0:1, 0:_MP], 0.0, _IMG_W)
    y1r = jnp.clip(merged[3][0:1, 0:_MP], 0.0, _IMG_H)
    x2r = jnp.clip(merged[4][0:1, 0:_MP], 0.0, _IMG_W)
    y2r = jnp.clip(merged[5][0:1, 0:_MP], 0.0, _IMG_H)

    lim = jax.lax.broadcasted_iota(i32, (_MP, _MP), 1)
    sim = jax.lax.broadcasted_iota(i32, (_MP, _MP), 0)
    eye = jnp.where(lim == sim, 1.0, 0.0)

    def to_col(row):
        return jnp.sum(eye * row, axis=1, keepdims=True)

    val_col = to_col(val_row)
    fidx_col = to_col(fidx_row)
    x1c, y1c, x2c, y2c = map(to_col, (x1r, y1r, x2r, y2r))

    bi_col = jnp.floor(fidx_col * 0.125)
    cls_col = fidx_col - 8.0 * bi_col
    cls_row = fidx_row - 8.0 * jnp.floor(fidx_row * 0.125)
    off = max(_IMG_W, _IMG_H) + 1.0
    ox1c, oy1c, ox2c, oy2c = (x1c + cls_col * off, y1c + cls_col * off,
                              x2c + cls_col * off, y2c + cls_col * off)
    ox1r, oy1r, ox2r, oy2r = (x1r + cls_row * off, y1r + cls_row * off,
                              x2r + cls_row * off, y2r + cls_row * off)

    area_c = jnp.maximum(ox2c - ox1c, 0.0) * jnp.maximum(oy2c - oy1c, 0.0)
    area_r = jnp.maximum(ox2r - ox1r, 0.0) * jnp.maximum(oy2r - oy1r, 0.0)
    iw = jnp.maximum(jnp.minimum(ox2c, ox2r) - jnp.maximum(ox1c, ox1r), 0.0)
    ih = jnp.maximum(jnp.minimum(oy2c, oy2r) - jnp.maximum(oy1c, oy1r), 0.0)
    inter = iw * ih
    iou = inter / jnp.maximum(area_c + area_r - inter, 1e-9)
    pgt = jnp.where((val_col > val_row)
                    | ((val_col == val_row) & (fidx_col < fidx_row)),
                    1.0, 0.0)
    a_ref[...] = jnp.where(iou > _NMS_THRESH, 1.0, 0.0) * pgt

    li1 = jax.lax.broadcasted_iota(i32, (1, _MP), 1).astype(f32)
    valid0 = jnp.where(li1 < nsurv, 1.0, 0.0)
    prank = jnp.sum(pgt * to_col(valid0), axis=0, keepdims=True)
    valid = valid0 * jnp.where(prank < float(_M), 1.0, 0.0)

    def nms_round(state):
        kept, und = state
        threat = jnp.sum(a_ref[...] * to_col(kept + und),
                         axis=0, keepdims=True)
        kthreat = jnp.sum(a_ref[...] * to_col(kept),
                          axis=0, keepdims=True)
        new_kept = und * jnp.where(threat == 0.0, 1.0, 0.0)
        new_dead = und * jnp.where(kthreat > 0.0, 1.0, 0.0)
        return kept + new_kept, und * (1.0 - new_kept) * (1.0 - new_dead)

    keep, _ = jax.lax.while_loop(lambda st: jnp.sum(st[1]) > 0.0, nms_round,
                                 (jnp.zeros_like(valid), valid))

    r_row = jnp.sum(pgt * to_col(keep), axis=0, keepdims=True)
    n_row = jnp.sum(pgt * to_col(valid - keep), axis=0, keepdims=True)
    tkept = jnp.sum(keep)
    osel = jnp.where(keep > 0.0, r_row,
                     jnp.where(valid > 0.0, tkept + n_row, 1e9))
    p_sub = jax.lax.broadcasted_iota(i32, (_PP, _MP), 0).astype(f32)
    wsel = jnp.where(osel == p_sub, 1.0, 0.0)
    hp = jax.lax.Precision.HIGHEST
    dsel = jnp.concatenate([x1c, y1c, x2c, y2c, val_col], axis=1)
    out5 = jnp.dot(wsel, dsel, precision=hp,
                   preferred_element_type=f32)
    bi_row = jnp.floor(fidx_row * 0.125)
    bi_sel = jnp.sum(wsel * bi_row, axis=1, keepdims=True)
    n_io = jax.lax.broadcasted_iota(i32, (_PP, _N), 1).astype(f32)
    wmot = jnp.where(n_io == bi_sel, 1.0, 0.0)
    mot_out = jnp.dot(wmot, motion_ref[...], precision=hp,
                      preferred_element_type=f32)
    out_ref[...] = jnp.concatenate([out5, mot_out], axis=1)[0:_TOPK, :]


def kernel(boxes, scores, mtype, morigin, maxis, mextrinsic):
    f32 = jnp.float32
    fgm = scores[:, :-1].reshape(_NC, _L).astype(f32)
    bc = boxes.reshape(_N, _K, 4)
    planes = [bc[:, :, c].reshape(_NC, _L).astype(f32) for c in range(4)]
    motion = jnp.concatenate([mtype, morigin, maxis, mextrinsic], axis=1)
    return pl.pallas_call(
        _mega_kernel,
        out_shape=jax.ShapeDtypeStruct((_TOPK, 26), f32),
        scratch_shapes=[pltpu.VMEM((_MP, _MP), f32)],
    )(fgm, *planes, motion)
